# baseline (speedup 1.0000x reference)
import jax
import jax.numpy as jnp
from jax import lax
from jax.experimental import pallas as pl
from jax.experimental.pallas import tpu as pltpu
from jax.experimental.pallas import tpu_sc as plsc

N = 10000
BATCH = 10
D = 128
E = 320000
E_ESP = 640000
H = 32
NB = N // BATCH
COUL = 1389.35457644382

KSUB = 16
ROW = 128
BIGC = KSUB * ROW
NW = 32

R_E = E // ROW
QE, RE_REM = divmod(R_E, NW)
KSUB_E = 8
BIGC_E = KSUB_E * ROW
NBIG_E = -(-(QE + 1) // KSUB_E)

R_P = E_ESP // ROW
QP, RP_REM = divmod(R_P, NW)
NBIG_P = -(-(QP + 1) // KSUB)

SD = 2 * D + H


def _dense_body(n_ref, w1_ref, b1_ref, w2_ref, out_ref, mono_ref):
    nb = n_ref[0]
    hs = jax.nn.silu(
        jnp.dot(nb, w1_ref[...], preferred_element_type=jnp.float32)
        + b1_ref[...])
    out8 = jnp.dot(hs, w2_ref[...], preferred_element_type=jnp.float32)
    q = out8[:, 2:3]
    mono_ref[0] = q - jnp.mean(q)
    qm = jnp.sum(out8[:, 3:4])
    ci = lax.broadcasted_iota(jnp.int32, (NB, 8), 1)
    out_ref[0] = jnp.where(ci == 3, qm, out8)


_dense_call = pl.pallas_call(
    _dense_body,
    grid=(BATCH,),
    in_specs=[
        pl.BlockSpec((1, NB, D), lambda b: (b, 0, 0)),
        pl.BlockSpec((D, SD), lambda b: (0, 0)),
        pl.BlockSpec((1, SD), lambda b: (0, 0)),
        pl.BlockSpec((SD, 8), lambda b: (0, 0)),
    ],
    out_specs=[
        pl.BlockSpec((1, NB, 8), lambda b: (b, 0, 0)),
        pl.BlockSpec((1, NB, 1), lambda b: (b, 0, 0)),
    ],
    out_shape=[
        jax.ShapeDtypeStruct((BATCH, NB, 8), jnp.float32),
        jax.ShapeDtypeStruct((BATCH, NB, 1), jnp.float32),
    ],
)


def _edge_body(s_hbm, r2_hbm, x_hbm, y_hbm, z_hbm, tab_hbm, parts_hbm,
               tab_v, s_v, r2_v, x_v, y_v, z_v, row8_v, shared,
               sem_lin, sem_sc):
    cid = lax.axis_index("c")
    sid = lax.axis_index("s")
    wid = sid * 2 + cid

    pltpu.sync_copy(tab_hbm, tab_v)

    lanes0 = lax.iota(jnp.int32, 16)
    zero16 = jnp.zeros((16,), jnp.float32)

    def zr(i, c_):
        plsc.store_scatter(row8_v, [jnp.full((16,), i, jnp.int32), lanes0],
                           zero16)
        return c_

    lax.fori_loop(0, KSUB_E * ROW, zr, 0)
    pltpu.sync_copy(row8_v.at[pl.ds(0, 625)],
                    shared.at[pl.ds(sid * 625, 625)])
    plsc.subcore_barrier()

    base = wid * QE + jnp.minimum(wid, RE_REM)
    hi = base + QE + jnp.where(wid < RE_REM, 1, 0)

    lin_bufs = (s_v, x_v, y_v, z_v)
    lin_srcs = (s_hbm, x_hbm, y_hbm, z_hbm)

    def chunk_start(g):
        row0 = base + g * KSUB_E
        return row0, jnp.minimum(row0, R_E - KSUB_E)

    def fire_linear(g, s):
        row0, srow = chunk_start(g)

        @pl.when(row0 < hi)
        def _():
            e0 = srow * ROW
            for src_h, dst in zip(lin_srcs, lin_bufs):
                pltpu.async_copy(src_h.at[pl.ds(e0, BIGC_E)],
                                 dst.at[s], sem_lin.at[s])
            pltpu.async_copy(r2_hbm.at[pl.ds(srow, KSUB_E)], r2_v.at[s],
                             sem_lin.at[s])

    def compute(g, s):
        row0, srow = chunk_start(g)

        @pl.when(row0 < hi)
        def _():
            off = row0 - srow
            nact = jnp.minimum(hi - row0, KSUB_E - off)
            for src_h, dst in zip(lin_srcs, lin_bufs):
                pltpu.make_async_copy(src_h.at[pl.ds(0, BIGC_E)],
                                      dst.at[s], sem_lin.at[s]).wait()
            pltpu.make_async_copy(r2_hbm.at[pl.ds(0, KSUB_E)], r2_v.at[s],
                                  sem_lin.at[s]).wait()

            def sub(jj, c2_):
                j = off + jj
                for v in range(ROW // 16):
                    lanes = lax.iota(jnp.int32, 16)
                    p = j * ROW + v * 16 + lanes
                    sv = plsc.load_gather(s_v.at[s], [p]) * 8
                    c1e = plsc.load_gather(tab_v, [sv])
                    c2e = plsc.load_gather(tab_v, [sv + 1])
                    x = plsc.load_gather(x_v.at[s], [p])
                    y = plsc.load_gather(y_v.at[s], [p])
                    z = plsc.load_gather(z_v.at[s], [p])
                    rr = jj * ROW + v * 16 + lanes
                    c2x = c2e * x
                    vals = (c1e * x, c1e * y, c1e * z,
                            c2x * x, c2e * y * y, c2e * z * z,
                            c2x * y, c2x * z, c2e * y * z)
                    for comp in range(9):
                        plsc.store_scatter(
                            row8_v, [rr, jnp.full((16,), comp, jnp.int32)],
                            vals[comp])
                pltpu.async_copy(row8_v.at[pl.ds(jj * ROW, ROW)],
                                 shared.at[r2_v.at[s].at[j]], sem_sc,
                                 add=True)
                return c2_

            lax.fori_loop(0, nact, sub, 0)

            def dr(jj, c2_):
                pltpu.make_async_copy(parts_hbm.at[0].at[pl.ds(0, ROW)],
                                      row8_v.at[pl.ds(0, ROW)],
                                      sem_sc).wait()
                return c2_

            lax.fori_loop(0, nact, dr, 0)

    fire_linear(0, 0)
    for g in range(NBIG_E):
        s = g % 2
        if g + 1 < NBIG_E:
            fire_linear(g + 1, 1 - s)
        compute(g, s)

    plsc.subcore_barrier()

    @pl.when(sid == 0)
    def _():
        pltpu.sync_copy(shared, parts_hbm.at[cid])


_edge_call = pl.kernel(
    _edge_body,
    out_type=jax.ShapeDtypeStruct((2, N, 16), jnp.float32),
    mesh=plsc.VectorSubcoreMesh(core_axis_name="c", subcore_axis_name="s"),
    compiler_params=pltpu.CompilerParams(
        needs_layout_passes=False, use_tc_tiling_on_sc=False),
    scratch_types=[
        pltpu.VMEM((N * 8,), jnp.float32),
        pltpu.VMEM((2, BIGC_E), jnp.int32),
        pltpu.VMEM((2, KSUB_E, ROW), jnp.int32),
        pltpu.VMEM((2, BIGC_E), jnp.float32),
        pltpu.VMEM((2, BIGC_E), jnp.float32),
        pltpu.VMEM((2, BIGC_E), jnp.float32),
        pltpu.VMEM((KSUB_E * ROW, 16), jnp.float32),
        pltpu.VMEM_SHARED((N, 16), jnp.float32),
        pltpu.SemaphoreType.DMA((2,)),
        pltpu.SemaphoreType.DMA,
    ],
)


def _esp_body(parts_hbm, mono_hbm, ri_hbm, qb_hbm, r1_hbm, mm_hbm,
              x_hbm, y_hbm, z_hbm, acc_hbm, table_hbm,
              mono_v, ri_v, qb_v, r1_v, mm_v, x_v, y_v, z_v, rows_v, acc_v,
              sem_lin, sem_ind):
    cid = lax.axis_index("c")
    sid = lax.axis_index("s")
    wid = sid * 2 + cid

    base = wid * QP + jnp.minimum(wid, RP_REM)
    hi = base + QP + jnp.where(wid < RP_REM, 1, 0)

    lin_bufs = (ri_v, qb_v, r1_v, mm_v, x_v, y_v, z_v)
    lin_srcs = (ri_hbm, qb_hbm, r1_hbm, mm_hbm, x_hbm, y_hbm, z_hbm)

    def chunk_start(g):
        row0 = base + g * KSUB
        return row0, jnp.minimum(row0, R_P - KSUB)

    def fire_linear(g, s):
        row0, srow = chunk_start(g)

        @pl.when(row0 < hi)
        def _():
            p0 = srow * ROW
            for src_h, dst in zip(lin_srcs, lin_bufs):
                pltpu.async_copy(src_h.at[pl.ds(p0, BIGC)],
                                 dst.at[s], sem_lin.at[s])

    fire_linear(0, 0)
    fire_linear(1, 1)

    r0 = sid * 625
    pltpu.async_copy(mono_hbm, mono_v, sem_ind.at[0])
    pltpu.async_copy(parts_hbm.at[0].at[pl.ds(r0, 625)],
                     rows_v.at[0].at[pl.ds(0, 625)], sem_ind.at[0])
    pltpu.async_copy(parts_hbm.at[1].at[pl.ds(r0, 625)],
                     rows_v.at[1].at[pl.ds(0, 625)], sem_ind.at[0])
    pltpu.make_async_copy(mono_hbm, mono_v, sem_ind.at[0]).wait()
    pltpu.make_async_copy(parts_hbm.at[0].at[pl.ds(0, 625)],
                          rows_v.at[0].at[pl.ds(0, 625)],
                          sem_ind.at[0]).wait()
    pltpu.make_async_copy(parts_hbm.at[1].at[pl.ds(0, 625)],
                          rows_v.at[1].at[pl.ds(0, 625)],
                          sem_ind.at[0]).wait()
    lanes0 = lax.iota(jnp.int32, 16)

    def mrg(i, c_):
        ridx = jnp.full((16,), i, jnp.int32)
        a = plsc.load_gather(rows_v.at[0], [ridx, lanes0])
        b = plsc.load_gather(rows_v.at[1], [ridx, lanes0])
        m = plsc.load_gather(mono_v, [jnp.full((16,), r0, jnp.int32) + i])
        plsc.store_scatter(rows_v.at[0], [ridx, lanes0],
                           jnp.where(lanes0 == 9, m, a + b))
        return c_

    lax.fori_loop(0, 625, mrg, 0)
    pltpu.sync_copy(rows_v.at[0].at[pl.ds(0, 625)],
                    table_hbm.at[pl.ds(r0, 625)])
    plsc.subcore_barrier()

    for b in range(BATCH):
        acc_v[pl.ds(b * 16, 16)] = jnp.zeros((16,), jnp.float32)

    def wait_linear_fire_indirect(g, s):
        row0, srow = chunk_start(g)

        @pl.when(row0 < hi)
        def _():
            for src_h, dst in zip(lin_srcs, lin_bufs):
                pltpu.make_async_copy(src_h.at[pl.ds(0, BIGC)],
                                      dst.at[s], sem_lin.at[s]).wait()

            def fire(j, c2_):
                pltpu.async_copy(
                    table_hbm.at[ri_v.at[s].at[pl.ds(j * ROW, ROW)]],
                    rows_v.at[s].at[pl.ds(j * ROW, ROW)], sem_ind.at[s])
                return c2_

            lax.fori_loop(0, KSUB, fire, 0)

    def compute(g, s):
        row0, srow = chunk_start(g)

        @pl.when(row0 < hi)
        def _():
            pltpu.make_async_copy(table_hbm.at[pl.ds(0, BIGC)],
                                  rows_v.at[s], sem_ind.at[s]).wait()

            def sub(j, c2_):
                row = srow + j

                @pl.when((row >= row0) & (row < hi))
                def _():
                    for v in range(ROW // 16):
                        lanes = lax.iota(jnp.int32, 16)
                        p = j * ROW + v * 16 + lanes

                        def cf(c):
                            return plsc.load_gather(
                                rows_v.at[s],
                                [p, jnp.full((16,), c, jnp.int32)])

                        dx, dy, dz = cf(0), cf(1), cf(2)
                        qxx, qyy, qzz = cf(3), cf(4), cf(5)
                        qxy, qxz, qyz = cf(6), cf(7), cf(8)
                        m = cf(9)
                        x = plsc.load_gather(x_v.at[s], [p])
                        y = plsc.load_gather(y_v.at[s], [p])
                        z = plsc.load_gather(z_v.at[s], [p])
                        r1 = plsc.load_gather(r1_v.at[s], [p])
                        mm = plsc.load_gather(mm_v.at[s], [p])
                        qb = plsc.load_gather(qb_v.at[s], [p])
                        r2 = r1 * r1
                        b0 = 1.0 / r1
                        b1 = b0 / r2
                        b2 = 3.0 * b1 / r2
                        g1 = dx * x + dy * y + dz * z
                        g2 = (qxx * x * x + qyy * y * y + qzz * z * z
                              + 2.0 * (qxy * x * y + qxz * x * z
                                       + qyz * y * z))
                        esp = (m * b0 + g1 * b1 + g2 * b2) * mm * COUL
                        plsc.addupdate_scatter(acc_v, [qb * 16 + lanes], esp)
                return c2_

            lax.fori_loop(0, KSUB, sub, 0)

    wait_linear_fire_indirect(0, 0)
    for g in range(NBIG_P):
        s = g % 2
        t = 1 - s
        if g + 1 < NBIG_P:
            wait_linear_fire_indirect(g + 1, t)
        compute(g, s)
        if g + 2 < NBIG_P:
            fire_linear(g + 2, s)

    pltpu.sync_copy(acc_v, acc_hbm.at[wid])


_esp_call = pl.kernel(
    _esp_body,
    out_type=[jax.ShapeDtypeStruct((NW, BATCH * 16), jnp.float32),
              jax.ShapeDtypeStruct((N, 16), jnp.float32)],
    mesh=plsc.VectorSubcoreMesh(core_axis_name="c", subcore_axis_name="s"),
    compiler_params=pltpu.CompilerParams(
        needs_layout_passes=False, use_tc_tiling_on_sc=False),
    scratch_types=[
        pltpu.VMEM((N,), jnp.float32),
        pltpu.VMEM((2, BIGC), jnp.int32),
        pltpu.VMEM((2, BIGC), jnp.int32),
        pltpu.VMEM((2, BIGC), jnp.float32),
        pltpu.VMEM((2, BIGC), jnp.float32),
        pltpu.VMEM((2, BIGC), jnp.float32),
        pltpu.VMEM((2, BIGC), jnp.float32),
        pltpu.VMEM((2, BIGC), jnp.float32),
        pltpu.VMEM((2, BIGC, 16), jnp.float32),
        pltpu.VMEM((BATCH * 16,), jnp.float32),
        pltpu.SemaphoreType.DMA((2,)),
        pltpu.SemaphoreType.DMA((2,)),
    ],
)


def kernel(nodes, senders, receivers, Rx1, R1_esp, Rx1_esp, mm_monos_esp,
           receivers_esp, qm_batch_esp,
           W_pot1, b_pot1, W_pot2, W_den1, b_den1, W_den2, W_c1, b_c1, W_c2):
    f32 = jnp.float32
    i32 = jnp.int32

    W1 = jnp.concatenate([W_pot1, W_den1, W_c1], axis=1)
    B1 = jnp.concatenate([b_pot1, b_den1, b_c1])[None, :]
    W2 = jnp.zeros((SD, 8), f32)
    W2 = W2.at[2 * D:, 0].set(W_c2[:, 1])
    W2 = W2.at[2 * D:, 1].set(W_c2[:, 2])
    W2 = W2.at[D:2 * D, 2].set(W_den2[:, 0] * 0.01)
    W2 = W2.at[:D, 3].set(W_pot2[:, 0])
    tab, mono = _dense_call(nodes.reshape(BATCH, NB, D), W1, B1, W2)
    qm_term = tab[:, 0, 3][:, None]

    parts = _edge_call(senders.astype(i32),
                       receivers.astype(i32).reshape(R_E, ROW),
                       Rx1[:, 0], Rx1[:, 1], Rx1[:, 2], tab.reshape(-1))

    acc, _ = _esp_call(parts, mono.reshape(-1), receivers_esp.astype(i32),
                       qm_batch_esp.astype(i32), R1_esp.reshape(-1),
                       mm_monos_esp.reshape(-1),
                       Rx1_esp[:, 0], Rx1_esp[:, 1], Rx1_esp[:, 2])

    coulomb = acc.reshape(NW, BATCH, 16).sum(axis=(0, 2))[:, None]
    return qm_term + coulomb


# speedup vs baseline: 1.0508x; 1.0508x over previous
import jax
import jax.numpy as jnp
from jax import lax
from jax.experimental import pallas as pl
from jax.experimental.pallas import tpu as pltpu
from jax.experimental.pallas import tpu_sc as plsc

N = 10000
BATCH = 10
D = 128
E = 320000
E_ESP = 640000
H = 32
NB = N // BATCH
COUL = 1389.35457644382

KSUB = 16
ROW = 128
BIGC = KSUB * ROW
NW = 32

R_E = E // ROW
QE, RE_REM = divmod(R_E, NW)
KSUB_E = 8
BIGC_E = KSUB_E * ROW
NBIG_E = -(-(QE + 1) // KSUB_E)

R_P = E_ESP // ROW
QP, RP_REM = divmod(R_P, NW)
NBIG_P = -(-(QP + 1) // KSUB)

SD = 2 * D + H


def _dense_body(n_ref, w1_ref, b1_ref, w2_ref, out_ref, mono_ref):
    nb = n_ref[0]
    hs = jax.nn.silu(
        jnp.dot(nb, w1_ref[...], preferred_element_type=jnp.float32)
        + b1_ref[...])
    out8 = jnp.dot(hs, w2_ref[...], preferred_element_type=jnp.float32)
    q = out8[:, 2:3]
    mono_ref[0] = q - jnp.mean(q)
    qm = jnp.sum(out8[:, 3:4])
    ci = lax.broadcasted_iota(jnp.int32, (NB, 8), 1)
    out_ref[0] = jnp.where(ci == 3, qm, out8)


_dense_call = pl.pallas_call(
    _dense_body,
    grid=(BATCH,),
    in_specs=[
        pl.BlockSpec((1, NB, D), lambda b: (b, 0, 0)),
        pl.BlockSpec((D, SD), lambda b: (0, 0)),
        pl.BlockSpec((1, SD), lambda b: (0, 0)),
        pl.BlockSpec((SD, 8), lambda b: (0, 0)),
    ],
    out_specs=[
        pl.BlockSpec((1, NB, 8), lambda b: (b, 0, 0)),
        pl.BlockSpec((1, NB, 1), lambda b: (b, 0, 0)),
    ],
    out_shape=[
        jax.ShapeDtypeStruct((BATCH, NB, 8), jnp.float32),
        jax.ShapeDtypeStruct((BATCH, NB, 1), jnp.float32),
    ],
)


def _edge_body(s_hbm, r2_hbm, x_hbm, y_hbm, z_hbm, tab_hbm, parts_hbm,
               tab_v, s_v, r2_v, x_v, y_v, z_v, row8_v, shared,
               sem_lin, sem_sc):
    cid = lax.axis_index("c")
    sid = lax.axis_index("s")
    wid = sid * 2 + cid

    pltpu.sync_copy(tab_hbm, tab_v)

    lanes0 = lax.iota(jnp.int32, 16)
    zero16 = jnp.zeros((16,), jnp.float32)

    def zr(i, c_):
        plsc.store_scatter(row8_v, [jnp.full((16,), i, jnp.int32), lanes0],
                           zero16)
        return c_

    lax.fori_loop(0, KSUB_E * ROW, zr, 0)
    pltpu.sync_copy(row8_v.at[pl.ds(0, 625)],
                    shared.at[pl.ds(sid * 625, 625)])
    plsc.subcore_barrier()

    base = wid * QE + jnp.minimum(wid, RE_REM)
    hi = base + QE + jnp.where(wid < RE_REM, 1, 0)

    lin_bufs = (s_v, x_v, y_v, z_v)
    lin_srcs = (s_hbm, x_hbm, y_hbm, z_hbm)

    def chunk_start(g):
        row0 = base + g * KSUB_E
        return row0, jnp.minimum(row0, R_E - KSUB_E)

    def fire_linear(g, s):
        row0, srow = chunk_start(g)

        @pl.when(row0 < hi)
        def _():
            e0 = srow * ROW
            for src_h, dst in zip(lin_srcs, lin_bufs):
                pltpu.async_copy(src_h.at[pl.ds(e0, BIGC_E)],
                                 dst.at[s], sem_lin.at[s])
            pltpu.async_copy(r2_hbm.at[pl.ds(srow, KSUB_E)], r2_v.at[s],
                             sem_lin.at[s])

    def compute(g, s):
        row0, srow = chunk_start(g)

        @pl.when(row0 < hi)
        def _():
            off = row0 - srow
            nact = jnp.minimum(hi - row0, KSUB_E - off)
            for src_h, dst in zip(lin_srcs, lin_bufs):
                pltpu.make_async_copy(src_h.at[pl.ds(0, BIGC_E)],
                                      dst.at[s], sem_lin.at[s]).wait()
            pltpu.make_async_copy(r2_hbm.at[pl.ds(0, KSUB_E)], r2_v.at[s],
                                  sem_lin.at[s]).wait()

            def sub(jj, c2_):
                j = off + jj
                for v in range(ROW // 16):
                    lanes = lax.iota(jnp.int32, 16)
                    p = j * ROW + v * 16 + lanes
                    sv = plsc.load_gather(s_v.at[s], [p]) * 8
                    c1e = plsc.load_gather(tab_v, [sv])
                    c2e = plsc.load_gather(tab_v, [sv + 1])
                    x = plsc.load_gather(x_v.at[s], [p])
                    y = plsc.load_gather(y_v.at[s], [p])
                    z = plsc.load_gather(z_v.at[s], [p])
                    rr = jj * ROW + v * 16 + lanes
                    c2x = c2e * x
                    vals = (c1e * x, c1e * y, c1e * z,
                            c2x * x, c2e * y * y, c2e * z * z,
                            c2x * y, c2x * z, c2e * y * z)
                    for comp in range(9):
                        plsc.store_scatter(
                            row8_v, [rr, jnp.full((16,), comp, jnp.int32)],
                            vals[comp])
                pltpu.async_copy(row8_v.at[pl.ds(jj * ROW, ROW)],
                                 shared.at[r2_v.at[s].at[j]], sem_sc,
                                 add=True)
                return c2_

            lax.fori_loop(0, nact, sub, 0)

            def dr(jj, c2_):
                pltpu.make_async_copy(parts_hbm.at[0].at[pl.ds(0, ROW)],
                                      row8_v.at[pl.ds(0, ROW)],
                                      sem_sc).wait()
                return c2_

            lax.fori_loop(0, nact, dr, 0)

    fire_linear(0, 0)
    for g in range(NBIG_E):
        s = g % 2
        if g + 1 < NBIG_E:
            fire_linear(g + 1, 1 - s)
        compute(g, s)

    plsc.subcore_barrier()

    @pl.when(sid == 0)
    def _():
        pltpu.sync_copy(shared, parts_hbm.at[cid])


_edge_call = pl.kernel(
    _edge_body,
    out_type=jax.ShapeDtypeStruct((2, N, 16), jnp.float32),
    mesh=plsc.VectorSubcoreMesh(core_axis_name="c", subcore_axis_name="s"),
    compiler_params=pltpu.CompilerParams(
        needs_layout_passes=False, use_tc_tiling_on_sc=False),
    scratch_types=[
        pltpu.VMEM((N * 8,), jnp.float32),
        pltpu.VMEM((2, BIGC_E), jnp.int32),
        pltpu.VMEM((2, KSUB_E, ROW), jnp.int32),
        pltpu.VMEM((2, BIGC_E), jnp.float32),
        pltpu.VMEM((2, BIGC_E), jnp.float32),
        pltpu.VMEM((2, BIGC_E), jnp.float32),
        pltpu.VMEM((KSUB_E * ROW, 16), jnp.float32),
        pltpu.VMEM_SHARED((N, 16), jnp.float32),
        pltpu.SemaphoreType.DMA((2,)),
        pltpu.SemaphoreType.DMA,
    ],
)


def _merge_body(p_ref, o_ref):
    o_ref[...] = p_ref[0] + p_ref[1]


_merge_call = pl.pallas_call(
    _merge_body,
    out_shape=jax.ShapeDtypeStruct((N * 16 // 128, 128), jnp.float32),
)


def _esp_body(table_hbm, mono_hbm, ri_hbm, qb_hbm, r1_hbm, mm_hbm,
              x_hbm, y_hbm, z_hbm, acc_hbm,
              mono_v, ri_v, qb_v, r1_v, mm_v, x_v, y_v, z_v, rows_v, acc_v,
              sem_lin, sem_ind):
    cid = lax.axis_index("c")
    sid = lax.axis_index("s")
    wid = sid * 2 + cid

    pltpu.sync_copy(mono_hbm, mono_v)
    for b in range(BATCH):
        acc_v[pl.ds(b * 16, 16)] = jnp.zeros((16,), jnp.float32)

    base = wid * QP + jnp.minimum(wid, RP_REM)
    hi = base + QP + jnp.where(wid < RP_REM, 1, 0)

    lin_bufs = (ri_v, qb_v, r1_v, mm_v, x_v, y_v, z_v)
    lin_srcs = (ri_hbm, qb_hbm, r1_hbm, mm_hbm, x_hbm, y_hbm, z_hbm)

    def chunk_start(g):
        row0 = base + g * KSUB
        return row0, jnp.minimum(row0, R_P - KSUB)

    def fire_linear(g, s):
        row0, srow = chunk_start(g)

        @pl.when(row0 < hi)
        def _():
            p0 = srow * ROW
            for src_h, dst in zip(lin_srcs, lin_bufs):
                pltpu.async_copy(src_h.at[pl.ds(p0, BIGC)],
                                 dst.at[s], sem_lin.at[s])

    def wait_linear_fire_indirect(g, s):
        row0, srow = chunk_start(g)

        @pl.when(row0 < hi)
        def _():
            for src_h, dst in zip(lin_srcs, lin_bufs):
                pltpu.make_async_copy(src_h.at[pl.ds(0, BIGC)],
                                      dst.at[s], sem_lin.at[s]).wait()

            def fire(j, c2_):
                pltpu.async_copy(
                    table_hbm.at[ri_v.at[s].at[pl.ds(j * ROW, ROW)]],
                    rows_v.at[s].at[pl.ds(j * ROW, ROW)], sem_ind.at[s])
                return c2_

            lax.fori_loop(0, KSUB, fire, 0)

    def compute(g, s):
        row0, srow = chunk_start(g)

        @pl.when(row0 < hi)
        def _():
            pltpu.make_async_copy(table_hbm.at[pl.ds(0, BIGC)],
                                  rows_v.at[s], sem_ind.at[s]).wait()

            def sub(j, c2_):
                row = srow + j

                @pl.when((row >= row0) & (row < hi))
                def _():
                    for v in range(ROW // 16):
                        lanes = lax.iota(jnp.int32, 16)
                        p = j * ROW + v * 16 + lanes
                        rv = plsc.load_gather(ri_v.at[s], [p])
                        m = plsc.load_gather(mono_v, [rv])

                        def cf(c):
                            return plsc.load_gather(
                                rows_v.at[s],
                                [p, jnp.full((16,), c, jnp.int32)])

                        dx, dy, dz = cf(0), cf(1), cf(2)
                        qxx, qyy, qzz = cf(3), cf(4), cf(5)
                        qxy, qxz, qyz = cf(6), cf(7), cf(8)
                        x = plsc.load_gather(x_v.at[s], [p])
                        y = plsc.load_gather(y_v.at[s], [p])
                        z = plsc.load_gather(z_v.at[s], [p])
                        r1 = plsc.load_gather(r1_v.at[s], [p])
                        mm = plsc.load_gather(mm_v.at[s], [p])
                        qb = plsc.load_gather(qb_v.at[s], [p])
                        r2 = r1 * r1
                        b0 = 1.0 / r1
                        b1 = b0 / r2
                        b2 = 3.0 * b1 / r2
                        g1 = dx * x + dy * y + dz * z
                        g2 = (qxx * x * x + qyy * y * y + qzz * z * z
                              + 2.0 * (qxy * x * y + qxz * x * z
                                       + qyz * y * z))
                        esp = (m * b0 + g1 * b1 + g2 * b2) * mm * COUL
                        plsc.addupdate_scatter(acc_v, [qb * 16 + lanes], esp)
                return c2_

            lax.fori_loop(0, KSUB, sub, 0)

    fire_linear(0, 0)
    fire_linear(1, 1)
    wait_linear_fire_indirect(0, 0)
    for g in range(NBIG_P):
        s = g % 2
        t = 1 - s
        if g + 1 < NBIG_P:
            wait_linear_fire_indirect(g + 1, t)
        compute(g, s)
        if g + 2 < NBIG_P:
            fire_linear(g + 2, s)

    pltpu.sync_copy(acc_v, acc_hbm.at[wid])


_esp_call = pl.kernel(
    _esp_body,
    out_type=jax.ShapeDtypeStruct((NW, BATCH * 16), jnp.float32),
    mesh=plsc.VectorSubcoreMesh(core_axis_name="c", subcore_axis_name="s"),
    compiler_params=pltpu.CompilerParams(
        needs_layout_passes=False, use_tc_tiling_on_sc=False),
    scratch_types=[
        pltpu.VMEM((N,), jnp.float32),
        pltpu.VMEM((2, BIGC), jnp.int32),
        pltpu.VMEM((2, BIGC), jnp.int32),
        pltpu.VMEM((2, BIGC), jnp.float32),
        pltpu.VMEM((2, BIGC), jnp.float32),
        pltpu.VMEM((2, BIGC), jnp.float32),
        pltpu.VMEM((2, BIGC), jnp.float32),
        pltpu.VMEM((2, BIGC), jnp.float32),
        pltpu.VMEM((2, BIGC, 16), jnp.float32),
        pltpu.VMEM((BATCH * 16,), jnp.float32),
        pltpu.SemaphoreType.DMA((2,)),
        pltpu.SemaphoreType.DMA((2,)),
    ],
)


def kernel(nodes, senders, receivers, Rx1, R1_esp, Rx1_esp, mm_monos_esp,
           receivers_esp, qm_batch_esp,
           W_pot1, b_pot1, W_pot2, W_den1, b_den1, W_den2, W_c1, b_c1, W_c2):
    f32 = jnp.float32
    i32 = jnp.int32

    W1 = jnp.concatenate([W_pot1, W_den1, W_c1], axis=1)
    B1 = jnp.concatenate([b_pot1, b_den1, b_c1])[None, :]
    W2 = jnp.zeros((SD, 8), f32)
    W2 = W2.at[2 * D:, 0].set(W_c2[:, 1])
    W2 = W2.at[2 * D:, 1].set(W_c2[:, 2])
    W2 = W2.at[D:2 * D, 2].set(W_den2[:, 0] * 0.01)
    W2 = W2.at[:D, 3].set(W_pot2[:, 0])
    tab, mono = _dense_call(nodes.reshape(BATCH, NB, D), W1, B1, W2)
    qm_term = tab[:, 0, 3][:, None]

    parts = _edge_call(senders.astype(i32),
                       receivers.astype(i32).reshape(R_E, ROW),
                       Rx1[:, 0], Rx1[:, 1], Rx1[:, 2], tab.reshape(-1))

    table = _merge_call(parts.reshape(2, N * 16 // 128, 128)).reshape(N, 16)

    acc = _esp_call(table, mono.reshape(-1), receivers_esp.astype(i32),
                    qm_batch_esp.astype(i32), R1_esp.reshape(-1),
                    mm_monos_esp.reshape(-1),
                    Rx1_esp[:, 0], Rx1_esp[:, 1], Rx1_esp[:, 2])

    coulomb = acc.reshape(NW, BATCH, 16).sum(axis=(0, 2))[:, None]
    return qm_term + coulomb


# speedup vs baseline: 1.1407x; 1.0855x over previous
import jax
import jax.numpy as jnp
from jax import lax
from jax.experimental import pallas as pl
from jax.experimental.pallas import tpu as pltpu
from jax.experimental.pallas import tpu_sc as plsc

N = 10000
BATCH = 10
D = 128
E = 320000
E_ESP = 640000
H = 32
NB = N // BATCH
COUL = 1389.35457644382

KSUB = 16
ROW = 128
BIGC = KSUB * ROW
NW = 32

R_E = E // ROW
QE, RE_REM = divmod(R_E, NW)
KSUB_E = 8
BIGC_E = KSUB_E * ROW
NBIG_E = -(-(QE + 1) // KSUB_E)

R_P = E_ESP // ROW
QP, RP_REM = divmod(R_P, NW)
NBIG_P = -(-(QP + 1) // KSUB)

SD = 2 * D + H


def _dense_body(n_ref, w1_ref, b1_ref, w2_ref, out_ref, mono_ref):
    nb = n_ref[0]
    hs = jax.nn.silu(
        jnp.dot(nb, w1_ref[...], preferred_element_type=jnp.float32)
        + b1_ref[...])
    out8 = jnp.dot(hs, w2_ref[...], preferred_element_type=jnp.float32)
    q = out8[:, 2:3]
    mono_ref[0] = q - jnp.mean(q)
    qm = jnp.sum(out8[:, 3:4])
    ci = lax.broadcasted_iota(jnp.int32, (NB, 8), 1)
    out_ref[0] = jnp.where(ci == 3, qm, out8)


_dense_call = pl.pallas_call(
    _dense_body,
    grid=(BATCH,),
    in_specs=[
        pl.BlockSpec((1, NB, D), lambda b: (b, 0, 0)),
        pl.BlockSpec((D, SD), lambda b: (0, 0)),
        pl.BlockSpec((1, SD), lambda b: (0, 0)),
        pl.BlockSpec((SD, 8), lambda b: (0, 0)),
    ],
    out_specs=[
        pl.BlockSpec((1, NB, 8), lambda b: (b, 0, 0)),
        pl.BlockSpec((1, NB, 1), lambda b: (b, 0, 0)),
    ],
    out_shape=[
        jax.ShapeDtypeStruct((BATCH, NB, 8), jnp.float32),
        jax.ShapeDtypeStruct((BATCH, NB, 1), jnp.float32),
    ],
)


def _edge_body(s_hbm, r2_hbm, x_hbm, y_hbm, z_hbm, tab_hbm, parts_hbm,
               tab_v, s_v, r2_v, x_v, y_v, z_v, row8_v, shared,
               sem_lin, sem_sc):
    cid = lax.axis_index("c")
    sid = lax.axis_index("s")
    wid = sid * 2 + cid

    pltpu.sync_copy(tab_hbm, tab_v)

    lanes0 = lax.iota(jnp.int32, 16)
    zero16 = jnp.zeros((16,), jnp.float32)

    def zr(i, c_):
        plsc.store_scatter(row8_v, [jnp.full((16,), i, jnp.int32), lanes0],
                           zero16)
        return c_

    lax.fori_loop(0, KSUB_E * ROW, zr, 0)
    pltpu.sync_copy(row8_v.at[pl.ds(0, 625)],
                    shared.at[pl.ds(sid * 625, 625)])
    plsc.subcore_barrier()

    base = wid * QE + jnp.minimum(wid, RE_REM)
    hi = base + QE + jnp.where(wid < RE_REM, 1, 0)

    lin_bufs = (s_v, x_v, y_v, z_v)
    lin_srcs = (s_hbm, x_hbm, y_hbm, z_hbm)

    def chunk_start(g):
        row0 = base + g * KSUB_E
        return row0, jnp.minimum(row0, R_E - KSUB_E)

    def fire_linear(g, s):
        row0, srow = chunk_start(g)

        @pl.when(row0 < hi)
        def _():
            e0 = srow * ROW
            for src_h, dst in zip(lin_srcs, lin_bufs):
                pltpu.async_copy(src_h.at[pl.ds(e0, BIGC_E)],
                                 dst.at[s], sem_lin.at[s])
            pltpu.async_copy(r2_hbm.at[pl.ds(srow, KSUB_E)], r2_v.at[s],
                             sem_lin.at[s])

    def compute(g, s):
        row0, srow = chunk_start(g)

        @pl.when(row0 < hi)
        def _():
            off = row0 - srow
            nact = jnp.minimum(hi - row0, KSUB_E - off)
            for src_h, dst in zip(lin_srcs, lin_bufs):
                pltpu.make_async_copy(src_h.at[pl.ds(0, BIGC_E)],
                                      dst.at[s], sem_lin.at[s]).wait()
            pltpu.make_async_copy(r2_hbm.at[pl.ds(0, KSUB_E)], r2_v.at[s],
                                  sem_lin.at[s]).wait()

            def sub(jj, c2_):
                j = off + jj
                for v in range(ROW // 16):
                    lanes = lax.iota(jnp.int32, 16)
                    p = j * ROW + v * 16 + lanes
                    sv = plsc.load_gather(s_v.at[s], [p]) * 8
                    c1e = plsc.load_gather(tab_v, [sv])
                    c2e = plsc.load_gather(tab_v, [sv + 1])
                    x = plsc.load_gather(x_v.at[s], [p])
                    y = plsc.load_gather(y_v.at[s], [p])
                    z = plsc.load_gather(z_v.at[s], [p])
                    rr = jj * ROW + v * 16 + lanes
                    c2x = c2e * x
                    vals = (c1e * x, c1e * y, c1e * z,
                            c2x * x, c2e * y * y, c2e * z * z,
                            c2x * y, c2x * z, c2e * y * z)
                    for comp in range(9):
                        plsc.store_scatter(
                            row8_v, [rr, jnp.full((16,), comp, jnp.int32)],
                            vals[comp])
                pltpu.async_copy(row8_v.at[pl.ds(jj * ROW, ROW)],
                                 shared.at[r2_v.at[s].at[j]], sem_sc,
                                 add=True)
                return c2_

            lax.fori_loop(0, nact, sub, 0)

            def dr(jj, c2_):
                pltpu.make_async_copy(parts_hbm.at[0].at[pl.ds(0, ROW)],
                                      row8_v.at[pl.ds(0, ROW)],
                                      sem_sc).wait()
                return c2_

            lax.fori_loop(0, nact, dr, 0)

    fire_linear(0, 0)
    for g in range(NBIG_E):
        s = g % 2
        if g + 1 < NBIG_E:
            fire_linear(g + 1, 1 - s)
        compute(g, s)

    plsc.subcore_barrier()

    @pl.when(sid == 0)
    def _():
        pltpu.sync_copy(shared, parts_hbm.at[cid])


_edge_call = pl.kernel(
    _edge_body,
    out_type=jax.ShapeDtypeStruct((2, N, 16), jnp.float32),
    mesh=plsc.VectorSubcoreMesh(core_axis_name="c", subcore_axis_name="s"),
    compiler_params=pltpu.CompilerParams(
        needs_layout_passes=False, use_tc_tiling_on_sc=False),
    scratch_types=[
        pltpu.VMEM((N * 8,), jnp.float32),
        pltpu.VMEM((2, BIGC_E), jnp.int32),
        pltpu.VMEM((2, KSUB_E, ROW), jnp.int32),
        pltpu.VMEM((2, BIGC_E), jnp.float32),
        pltpu.VMEM((2, BIGC_E), jnp.float32),
        pltpu.VMEM((2, BIGC_E), jnp.float32),
        pltpu.VMEM((KSUB_E * ROW, 16), jnp.float32),
        pltpu.VMEM_SHARED((N, 16), jnp.float32),
        pltpu.SemaphoreType.DMA((2,)),
        pltpu.SemaphoreType.DMA,
    ],
)


def _merge_body(p_ref, o_ref):
    o_ref[...] = p_ref[0] + p_ref[1]


_merge_call = pl.pallas_call(
    _merge_body,
    out_shape=jax.ShapeDtypeStruct((N * 16 // 128, 128), jnp.float32),
)


def _esp_body(table_hbm, mono_hbm, ri_hbm, qb_hbm, r1_hbm, mm_hbm,
              x_hbm, y_hbm, z_hbm, acc_hbm,
              mono_v, ri_v, qb_v, r1_v, mm_v, x_v, y_v, z_v, rows_v, acc_v,
              sem_lin, sem_ind):
    cid = lax.axis_index("c")
    sid = lax.axis_index("s")
    wid = sid * 2 + cid

    pltpu.sync_copy(mono_hbm, mono_v)
    for b in range(BATCH):
        acc_v[pl.ds(b * 16, 16)] = jnp.zeros((16,), jnp.float32)

    base = wid * QP + jnp.minimum(wid, RP_REM)
    hi = base + QP + jnp.where(wid < RP_REM, 1, 0)

    lin_bufs = (ri_v, qb_v, r1_v, mm_v, x_v, y_v, z_v)
    lin_srcs = (ri_hbm, qb_hbm, r1_hbm, mm_hbm, x_hbm, y_hbm, z_hbm)

    def chunk_start(g):
        row0 = base + g * KSUB
        return row0, jnp.minimum(row0, R_P - KSUB)

    def fire_linear(g, s):
        row0, srow = chunk_start(g)

        @pl.when(row0 < hi)
        def _():
            p0 = srow * ROW
            for src_h, dst in zip(lin_srcs, lin_bufs):
                pltpu.async_copy(src_h.at[pl.ds(p0, BIGC)],
                                 dst.at[s], sem_lin.at[s])

    def wait_linear_fire_indirect(g, s):
        row0, srow = chunk_start(g)

        @pl.when(row0 < hi)
        def _():
            for src_h, dst in zip(lin_srcs, lin_bufs):
                pltpu.make_async_copy(src_h.at[pl.ds(0, BIGC)],
                                      dst.at[s], sem_lin.at[s]).wait()

            def fire(j, c2_):
                pltpu.async_copy(
                    table_hbm.at[ri_v.at[s].at[pl.ds(j * ROW, ROW)]],
                    rows_v.at[s].at[pl.ds(j * ROW, ROW)], sem_ind.at[s])
                return c2_

            lax.fori_loop(0, KSUB, fire, 0)

    def compute(g, s):
        row0, srow = chunk_start(g)

        @pl.when(row0 < hi)
        def _():
            pltpu.make_async_copy(table_hbm.at[pl.ds(0, BIGC)],
                                  rows_v.at[s], sem_ind.at[s]).wait()

            def sub(j, c2_):
                row = srow + j

                @pl.when((row >= row0) & (row < hi))
                def _():
                    pend = []
                    for v in range(ROW // 16):
                        lanes = lax.iota(jnp.int32, 16)
                        p = j * ROW + v * 16 + lanes
                        rv = plsc.load_gather(ri_v.at[s], [p])
                        m = plsc.load_gather(mono_v, [rv])

                        def cf(c):
                            return plsc.load_gather(
                                rows_v.at[s],
                                [p, jnp.full((16,), c, jnp.int32)])

                        dx, dy, dz = cf(0), cf(1), cf(2)
                        qxx, qyy, qzz = cf(3), cf(4), cf(5)
                        qxy, qxz, qyz = cf(6), cf(7), cf(8)
                        x = plsc.load_gather(x_v.at[s], [p])
                        y = plsc.load_gather(y_v.at[s], [p])
                        z = plsc.load_gather(z_v.at[s], [p])
                        r1 = plsc.load_gather(r1_v.at[s], [p])
                        mm = plsc.load_gather(mm_v.at[s], [p])
                        qb = plsc.load_gather(qb_v.at[s], [p])
                        r2 = r1 * r1
                        b0 = 1.0 / r1
                        b1 = b0 / r2
                        b2 = 3.0 * b1 / r2
                        g1 = dx * x + dy * y + dz * z
                        g2 = (qxx * x * x + qyy * y * y + qzz * z * z
                              + 2.0 * (qxy * x * y + qxz * x * z
                                       + qyz * y * z))
                        esp = (m * b0 + g1 * b1 + g2 * b2) * mm * COUL
                        pend.append((qb * 16 + lanes, esp))
                    for addr, val in pend:
                        plsc.addupdate_scatter(acc_v, [addr], val)
                return c2_

            lax.fori_loop(0, KSUB, sub, 0)

    fire_linear(0, 0)
    fire_linear(1, 1)
    wait_linear_fire_indirect(0, 0)
    for g in range(NBIG_P):
        s = g % 2
        t = 1 - s
        if g + 1 < NBIG_P:
            wait_linear_fire_indirect(g + 1, t)
        compute(g, s)
        if g + 2 < NBIG_P:
            fire_linear(g + 2, s)

    pltpu.sync_copy(acc_v, acc_hbm.at[wid])


_esp_call = pl.kernel(
    _esp_body,
    out_type=jax.ShapeDtypeStruct((NW, BATCH * 16), jnp.float32),
    mesh=plsc.VectorSubcoreMesh(core_axis_name="c", subcore_axis_name="s"),
    compiler_params=pltpu.CompilerParams(
        needs_layout_passes=False, use_tc_tiling_on_sc=False),
    scratch_types=[
        pltpu.VMEM((N,), jnp.float32),
        pltpu.VMEM((2, BIGC), jnp.int32),
        pltpu.VMEM((2, BIGC), jnp.int32),
        pltpu.VMEM((2, BIGC), jnp.float32),
        pltpu.VMEM((2, BIGC), jnp.float32),
        pltpu.VMEM((2, BIGC), jnp.float32),
        pltpu.VMEM((2, BIGC), jnp.float32),
        pltpu.VMEM((2, BIGC), jnp.float32),
        pltpu.VMEM((2, BIGC, 16), jnp.float32),
        pltpu.VMEM((BATCH * 16,), jnp.float32),
        pltpu.SemaphoreType.DMA((2,)),
        pltpu.SemaphoreType.DMA((2,)),
    ],
)


def kernel(nodes, senders, receivers, Rx1, R1_esp, Rx1_esp, mm_monos_esp,
           receivers_esp, qm_batch_esp,
           W_pot1, b_pot1, W_pot2, W_den1, b_den1, W_den2, W_c1, b_c1, W_c2):
    f32 = jnp.float32
    i32 = jnp.int32

    W1 = jnp.concatenate([W_pot1, W_den1, W_c1], axis=1)
    B1 = jnp.concatenate([b_pot1, b_den1, b_c1])[None, :]
    W2 = jnp.zeros((SD, 8), f32)
    W2 = W2.at[2 * D:, 0].set(W_c2[:, 1])
    W2 = W2.at[2 * D:, 1].set(W_c2[:, 2])
    W2 = W2.at[D:2 * D, 2].set(W_den2[:, 0] * 0.01)
    W2 = W2.at[:D, 3].set(W_pot2[:, 0])
    tab, mono = _dense_call(nodes.reshape(BATCH, NB, D), W1, B1, W2)
    qm_term = tab[:, 0, 3][:, None]

    parts = _edge_call(senders.astype(i32),
                       receivers.astype(i32).reshape(R_E, ROW),
                       Rx1[:, 0], Rx1[:, 1], Rx1[:, 2], tab.reshape(-1))

    table = _merge_call(parts.reshape(2, N * 16 // 128, 128)).reshape(N, 16)

    acc = _esp_call(table, mono.reshape(-1), receivers_esp.astype(i32),
                    qm_batch_esp.astype(i32), R1_esp.reshape(-1),
                    mm_monos_esp.reshape(-1),
                    Rx1_esp[:, 0], Rx1_esp[:, 1], Rx1_esp[:, 2])

    coulomb = acc.reshape(NW, BATCH, 16).sum(axis=(0, 2))[:, None]
    return qm_term + coulomb
